# baseline (speedup 1.0000x reference)
import functools

import jax
import jax.numpy as jnp
from jax import lax
from jax.experimental import pallas as pl
from jax.experimental.pallas import tpu as pltpu
from jax.experimental.pallas import tpu_sc as plsc

_N_DIM = 32
_R = 10.0
_T = 1.0

_INFO = plsc.get_sparse_core_info()
_NC = _INFO.num_cores
_NS = _INFO.num_subcores
_NW = _NC * _NS
_CHUNK = 128


def _sc_gather(flat_idx, table, n_idx):
  b_per_w = n_idx // _NW
  n_chunks = b_per_w // _CHUNK
  mesh = plsc.VectorSubcoreMesh(core_axis_name="c", subcore_axis_name="s")

  def body(idx_hbm, table_hbm, out_hbm, idx_v, rows_v, sem):
    wid = lax.axis_index("s") * _NC + lax.axis_index("c")
    base = wid * b_per_w
    pltpu.sync_copy(idx_hbm.at[wid], idx_v)
    copies = []
    for k in range(n_chunks):
      copies.append(
          pltpu.async_copy(
              table_hbm.at[idx_v.at[k]],
              rows_v.at[pl.ds(k * _CHUNK, _CHUNK)],
              sem,
          ))
    for c in copies:
      c.wait()
    pltpu.sync_copy(rows_v, out_hbm.at[pl.ds(base, b_per_w)])

  return pl.kernel(
      body,
      out_type=jax.ShapeDtypeStruct((n_idx, _N_DIM), jnp.float32),
      mesh=mesh,
      compiler_params=pltpu.CompilerParams(use_tc_tiling_on_sc=False),
      scratch_types=[
          pltpu.VMEM((n_chunks, _CHUNK), jnp.int32),
          pltpu.VMEM((b_per_w, _N_DIM), jnp.float32),
          pltpu.SemaphoreType.DMA,
      ],
  )(flat_idx.reshape(_NW, n_chunks, _CHUNK), table)


def _loss_body(x_ref, lab_ref, o_ref):
  x = x_ref[...]
  u = x[:, :_N_DIM]
  v = x[:, _N_DIM:]
  d2 = jnp.sum((u - v) ** 2, axis=1, keepdims=True)
  nu = jnp.sum(u * u, axis=1, keepdims=True)
  nv = jnp.sum(v * v, axis=1, keepdims=True)
  ret = 1.0 + 2.0 * d2 / ((1.0 - nu) * (1.0 - nv))
  dist = jnp.log(ret + jnp.sqrt(ret * ret - 1.0))
  z = (dist - _R) / _T
  labf = lab_ref[...].astype(jnp.float32)
  loss = jnp.where(labf == 1.0,
                   jnp.log(jnp.exp(z) + 1.0),
                   jnp.log(1.0 + jnp.exp(-z)))
  o_ref[...] = loss


def kernel(pairs, labels, table):
  batch = pairs.shape[0]
  flat_idx = pairs.reshape(2 * batch)
  rows = _sc_gather(flat_idx, table, 2 * batch)
  x = rows.reshape(batch, 2 * _N_DIM)
  lab2 = labels.reshape(batch, 1)
  out = pl.pallas_call(
      _loss_body,
      out_shape=jax.ShapeDtypeStruct((batch, 1), jnp.float32),
  )(x, lab2)
  return out.reshape(batch)
